# baseline (speedup 1.0000x reference)
import functools

import jax
import jax.numpy as jnp
from jax import lax
from jax.experimental import pallas as pl
from jax.experimental.pallas import tpu as pltpu
from jax.experimental.pallas import tpu_sc as plsc

B, S, D = 4, 4096, 1024
N = B * S
NC, NS = 2, 16
NW = NC * NS
ROWS_PER_W = N // NW
CHUNK = 32
N_CHUNKS = ROWS_PER_W // CHUNK
LANES = 16


def _emb_body(tokens_hbm, tok_table_hbm, pos_table_hbm, out_hbm,
              idx_v, tokbuf, posbuf, sem):
    wid = lax.axis_index("s") * NC + lax.axis_index("c")
    base = wid * ROWS_PER_W
    s0 = base % S
    pltpu.sync_copy(tokens_hbm.at[pl.ds(base, ROWS_PER_W)], idx_v)

    def chunk_body(ci, carry):
        cp = pltpu.async_copy(
            tok_table_hbm.at[idx_v.at[pl.ds(ci * CHUNK, CHUNK)]], tokbuf, sem)
        pltpu.sync_copy(pos_table_hbm.at[pl.ds(s0 + ci * CHUNK, CHUNK)], posbuf)
        cp.wait()

        def row_body(r, c2):
            for col in range(D // LANES):
                sl = pl.ds(col * LANES, LANES)
                plsc.addupdate(posbuf.at[r, sl], tokbuf[r, sl])
            return c2

        lax.fori_loop(0, CHUNK, row_body, 0)
        pltpu.sync_copy(posbuf, out_hbm.at[pl.ds(base + ci * CHUNK, CHUNK)])
        return carry

    lax.fori_loop(0, N_CHUNKS, chunk_body, 0)


@jax.jit
def _emb(tokens_flat, token_table, pos_table):
    mesh = plsc.VectorSubcoreMesh(core_axis_name="c", subcore_axis_name="s")
    kern = functools.partial(
        pl.kernel,
        mesh=mesh,
        out_type=jax.ShapeDtypeStruct((N, D), jnp.float32),
        scratch_types=[
            pltpu.VMEM((ROWS_PER_W,), jnp.int32),
            pltpu.VMEM((CHUNK, D), jnp.float32),
            pltpu.VMEM((CHUNK, D), jnp.float32),
            pltpu.SemaphoreType.DMA,
        ],
    )(_emb_body)
    return kern(tokens_flat, token_table, pos_table)


def kernel(tokens, token_table, pos_table):
    out = _emb(tokens.reshape(N).astype(jnp.int32), token_table, pos_table)
    return out.reshape(B, S, D)


# speedup vs baseline: 1.4899x; 1.4899x over previous
import functools

import jax
import jax.numpy as jnp
from jax import lax
from jax.experimental import pallas as pl
from jax.experimental.pallas import tpu as pltpu
from jax.experimental.pallas import tpu_sc as plsc

B, S, D = 4, 4096, 1024
N = B * S
NC, NS = 2, 16
NW = NC * NS
ROWS_PER_W = N // NW
CHUNK = 8
N_CHUNKS = ROWS_PER_W // CHUNK
NBUF = 4
N_GROUPS = N_CHUNKS // NBUF
LANES = 16


def _emb_body(tokens_hbm, tok_table_hbm, pos_table_hbm, out_hbm, idx_v,
              *rest):
    tokbufs = rest[:NBUF]
    posbufs = rest[NBUF:2 * NBUF]
    tok_sems = rest[2 * NBUF:3 * NBUF]
    pos_sems = rest[3 * NBUF:4 * NBUF]
    wb_sems = rest[4 * NBUF:5 * NBUF]

    wid = lax.axis_index("s") * NC + lax.axis_index("c")
    base = wid * ROWS_PER_W
    s0 = base % S
    pltpu.sync_copy(tokens_hbm.at[pl.ds(base, ROWS_PER_W)], idx_v)

    def issue_loads(ci, k):
        pltpu.async_copy(tok_table_hbm.at[idx_v.at[pl.ds(ci * CHUNK, CHUNK)]],
                         tokbufs[k], tok_sems[k])
        pltpu.async_copy(pos_table_hbm.at[pl.ds(s0 + ci * CHUNK, CHUNK)],
                         posbufs[k], pos_sems[k])

    def wait_loads(ci, k):
        pltpu.make_async_copy(
            tok_table_hbm.at[idx_v.at[pl.ds(ci * CHUNK, CHUNK)]],
            tokbufs[k], tok_sems[k]).wait()
        pltpu.make_async_copy(pos_table_hbm.at[pl.ds(s0, CHUNK)],
                              posbufs[k], pos_sems[k]).wait()

    def add_chunk(k):
        def row_body(r, c2):
            for col in range(D // LANES):
                sl = pl.ds(col * LANES, LANES)
                plsc.addupdate(tokbufs[k].at[r, sl], posbufs[k][r, sl])
            return c2
        lax.fori_loop(0, CHUNK, row_body, 0)

    def issue_wb(ci, k):
        pltpu.async_copy(tokbufs[k],
                         out_hbm.at[pl.ds(base + ci * CHUNK, CHUNK)],
                         wb_sems[k])

    def drain_wb(ci, k):
        pltpu.make_async_copy(tokbufs[k],
                              out_hbm.at[pl.ds(base + ci * CHUNK, CHUNK)],
                              wb_sems[k]).wait()

    AHEAD = 2

    def slot(ci, k, first_group, last_group):
        if not first_group:
            drain_wb(ci - AHEAD, (k - AHEAD) % NBUF)
        if not last_group:
            issue_loads(ci + AHEAD, (k + AHEAD) % NBUF)
        wait_loads(ci, k)
        add_chunk(k)
        issue_wb(ci, k)

    for k in range(AHEAD):
        issue_loads(k, k)
    for k in range(NBUF):
        slot(k, k, first_group=(k < AHEAD), last_group=False)

    def group_body(g, carry):
        c0 = g * NBUF
        for k in range(NBUF):
            slot(c0 + k, k, first_group=False, last_group=False)
        return carry

    lax.fori_loop(1, N_GROUPS - 1, group_body, 0)

    c0 = (N_GROUPS - 1) * NBUF
    for k in range(NBUF):
        slot(c0 + k, k, first_group=False,
             last_group=(k >= NBUF - AHEAD))
    for k in range(NBUF - AHEAD, NBUF):
        drain_wb(c0 + k, k)


@jax.jit
def _emb(tokens_flat, token_table, pos_table):
    mesh = plsc.VectorSubcoreMesh(core_axis_name="c", subcore_axis_name="s")
    scratch = [pltpu.VMEM((ROWS_PER_W,), jnp.int32)]
    scratch += [pltpu.VMEM((CHUNK, D), jnp.float32) for _ in range(2 * NBUF)]
    scratch += [pltpu.SemaphoreType.DMA for _ in range(3 * NBUF)]
    kern = functools.partial(
        pl.kernel,
        mesh=mesh,
        out_type=jax.ShapeDtypeStruct((N, D), jnp.float32),
        scratch_types=scratch,
    )(_emb_body)
    return kern(tokens_flat, token_table, pos_table)


def kernel(tokens, token_table, pos_table):
    out = _emb(tokens.reshape(N).astype(jnp.int32), token_table, pos_table)
    return out.reshape(B, S, D)


# speedup vs baseline: 1.6855x; 1.1313x over previous
import functools

import jax
import jax.numpy as jnp
from jax import lax
from jax.experimental import pallas as pl
from jax.experimental.pallas import tpu as pltpu
from jax.experimental.pallas import tpu_sc as plsc

B, S, D = 4, 4096, 1024
NC, NS = 2, 16
NW = NC * NS
PPW = S // NW
CH = 16
NCH = PPW // CH
NSLOT = NCH * B
NBUF = 4
LANES = 16


def _emb_body(tokens_hbm, tok_table_hbm, pos_table_hbm, out_hbm, idx_v,
              *rest):
    tokbufs = rest[0:NBUF]
    posbufs = rest[NBUF:NBUF + 2]
    tok_sems = rest[NBUF + 2:2 * NBUF + 2]
    pos_sems = rest[2 * NBUF + 2:2 * NBUF + 4]
    wb_sems = rest[2 * NBUF + 4:3 * NBUF + 4]
    idx_sem = rest[3 * NBUF + 4]

    wid = lax.axis_index("s") * NC + lax.axis_index("c")
    p0 = wid * PPW

    idx_cps = [
        pltpu.async_copy(tokens_hbm.at[b, pl.ds(p0, PPW)],
                         idx_v.at[pl.ds(b * PPW, PPW)], idx_sem)
        for b in range(B)
    ]
    for cp in idx_cps:
        cp.wait()

    def tok_pair(t):
        ci, b = t // B, t % B
        k = t % NBUF
        src = tok_table_hbm.at[idx_v.at[pl.ds(b * PPW + ci * CH, CH)]]
        return src, tokbufs[k], tok_sems[k]

    def issue_tok(t):
        src, dst, sem = tok_pair(t)
        pltpu.async_copy(src, dst, sem)

    def wait_tok(t):
        src, dst, sem = tok_pair(t)
        pltpu.make_async_copy(src, dst, sem).wait()

    def pos_pair(ci):
        j = ci % 2
        return (pos_table_hbm.at[pl.ds(p0 + ci * CH, CH)], posbufs[j],
                pos_sems[j])

    def issue_pos(ci):
        src, dst, sem = pos_pair(ci)
        pltpu.async_copy(src, dst, sem)

    def wait_pos(ci):
        src, dst, sem = pos_pair(ci)
        pltpu.make_async_copy(src, dst, sem).wait()

    def wb_pair(t):
        ci, b = t // B, t % B
        k = t % NBUF
        return (tokbufs[k], out_hbm.at[pl.ds(b * S + p0 + ci * CH, CH)],
                wb_sems[k])

    def issue_wb(t):
        src, dst, sem = wb_pair(t)
        pltpu.async_copy(src, dst, sem)

    def drain_wb(t):
        src, dst, sem = wb_pair(t)
        pltpu.make_async_copy(src, dst, sem).wait()

    def add_chunk(t):
        ci = t // B
        tok, pos = tokbufs[t % NBUF], posbufs[ci % 2]

        def row_body(r, c2):
            for col in range(D // LANES):
                sl = pl.ds(col * LANES, LANES)
                plsc.addupdate(tok.at[r, sl], pos[r, sl])
            return c2

        lax.fori_loop(0, CH, row_body, 0)

    issue_pos(0)
    issue_tok(0)
    issue_tok(1)

    for t in range(NSLOT):
        ci, b = t // B, t % B
        if t >= 2:
            drain_wb(t - 2)
        if t + 2 < NSLOT:
            issue_tok(t + 2)
        if b == 0:
            if ci + 1 < NCH:
                issue_pos(ci + 1)
            wait_pos(ci)
        wait_tok(t)
        add_chunk(t)
        issue_wb(t)

    drain_wb(NSLOT - 2)
    drain_wb(NSLOT - 1)


@jax.jit
def _emb(tokens, token_table, pos_table):
    mesh = plsc.VectorSubcoreMesh(core_axis_name="c", subcore_axis_name="s")
    scratch = [pltpu.VMEM((B * PPW,), jnp.int32)]
    scratch += [pltpu.VMEM((CH, D), jnp.float32) for _ in range(NBUF + 2)]
    scratch += [pltpu.SemaphoreType.DMA for _ in range(2 * NBUF + 3)]
    kern = functools.partial(
        pl.kernel,
        mesh=mesh,
        out_type=jax.ShapeDtypeStruct((B * S, D), jnp.float32),
        scratch_types=scratch,
    )(_emb_body)
    return kern(tokens, token_table, pos_table)


def kernel(tokens, token_table, pos_table):
    out = _emb(tokens.astype(jnp.int32), token_table, pos_table)
    return out.reshape(B, S, D)


# speedup vs baseline: 1.7004x; 1.0088x over previous
import functools

import jax
import jax.numpy as jnp
from jax import lax
from jax.experimental import pallas as pl
from jax.experimental.pallas import tpu as pltpu
from jax.experimental.pallas import tpu_sc as plsc

B, S, D = 4, 4096, 1024
NC, NS = 2, 16
NW = NC * NS
PPW = S // NW
CH = 16
NCH = PPW // CH
NSLOT = NCH * B
NBUF = 5
AHEAD = 3
LANES = 16


def _emb_body(tokens_hbm, tok_table_hbm, pos_table_hbm, out_hbm, idx_v,
              *rest):
    tokbufs = rest[0:NBUF]
    posbufs = rest[NBUF:NBUF + 2]
    tok_sems = rest[NBUF + 2:2 * NBUF + 2]
    pos_sems = rest[2 * NBUF + 2:2 * NBUF + 4]
    wb_sems = rest[2 * NBUF + 4:3 * NBUF + 4]
    idx_sem = rest[3 * NBUF + 4]

    wid = lax.axis_index("s") * NC + lax.axis_index("c")
    p0 = wid * PPW

    idx_cps = [
        pltpu.async_copy(tokens_hbm.at[b, pl.ds(p0, PPW)],
                         idx_v.at[pl.ds(b * PPW, PPW)], idx_sem)
        for b in range(B)
    ]
    for cp in idx_cps:
        cp.wait()

    def tok_pair(t):
        ci, b = t // B, t % B
        k = t % NBUF
        src = tok_table_hbm.at[idx_v.at[pl.ds(b * PPW + ci * CH, CH)]]
        return src, tokbufs[k], tok_sems[k]

    def issue_tok(t):
        src, dst, sem = tok_pair(t)
        pltpu.async_copy(src, dst, sem)

    def wait_tok(t):
        src, dst, sem = tok_pair(t)
        pltpu.make_async_copy(src, dst, sem).wait()

    def pos_pair(ci):
        j = ci % 2
        return (pos_table_hbm.at[pl.ds(p0 + ci * CH, CH)], posbufs[j],
                pos_sems[j])

    def issue_pos(ci):
        src, dst, sem = pos_pair(ci)
        pltpu.async_copy(src, dst, sem)

    def wait_pos(ci):
        src, dst, sem = pos_pair(ci)
        pltpu.make_async_copy(src, dst, sem).wait()

    def wb_pair(t):
        ci, b = t // B, t % B
        k = t % NBUF
        return (tokbufs[k], out_hbm.at[pl.ds(b * S + p0 + ci * CH, CH)],
                wb_sems[k])

    def issue_wb(t):
        src, dst, sem = wb_pair(t)
        pltpu.async_copy(src, dst, sem)

    def drain_wb(t):
        src, dst, sem = wb_pair(t)
        pltpu.make_async_copy(src, dst, sem).wait()

    def add_chunk(t):
        ci = t // B
        tok, pos = tokbufs[t % NBUF], posbufs[ci % 2]
        ncol = D // LANES

        def row_body(r, c2):
            prev = pos[r, pl.ds(0, LANES)]
            for col in range(1, ncol):
                cur = pos[r, pl.ds(col * LANES, LANES)]
                plsc.addupdate(tok.at[r, pl.ds((col - 1) * LANES, LANES)],
                               prev)
                prev = cur
            plsc.addupdate(tok.at[r, pl.ds((ncol - 1) * LANES, LANES)], prev)
            return c2

        lax.fori_loop(0, CH, row_body, 0)

    issue_pos(0)
    for t in range(AHEAD):
        issue_tok(t)

    for t in range(NSLOT):
        ci, b = t // B, t % B
        if t >= NBUF - AHEAD:
            drain_wb(t - (NBUF - AHEAD))
        if t + AHEAD < NSLOT:
            issue_tok(t + AHEAD)
        if b == 0:
            if ci + 1 < NCH:
                issue_pos(ci + 1)
            wait_pos(ci)
        wait_tok(t)
        add_chunk(t)
        issue_wb(t)

    for t in range(NSLOT - (NBUF - AHEAD), NSLOT):
        drain_wb(t)


@jax.jit
def _emb(tokens, token_table, pos_table):
    mesh = plsc.VectorSubcoreMesh(core_axis_name="c", subcore_axis_name="s")
    scratch = [pltpu.VMEM((B * PPW,), jnp.int32)]
    scratch += [pltpu.VMEM((CH, D), jnp.float32) for _ in range(NBUF + 2)]
    scratch += [pltpu.SemaphoreType.DMA for _ in range(2 * NBUF + 3)]
    kern = functools.partial(
        pl.kernel,
        mesh=mesh,
        out_type=jax.ShapeDtypeStruct((B * S, D), jnp.float32),
        scratch_types=scratch,
    )(_emb_body)
    return kern(tokens, token_table, pos_table)


def kernel(tokens, token_table, pos_table):
    out = _emb(tokens.astype(jnp.int32), token_table, pos_table)
    return out.reshape(B, S, D)
